# Initial kernel scaffold; baseline (speedup 1.0000x reference)
import jax
import jax.numpy as jnp
from jax.experimental import pallas as pl


def kernel(x, W):
    raise NotImplementedError("write your pallas kernel here")


# speedup vs baseline: 4.8089x; 4.8089x over previous
import functools

import jax
import jax.numpy as jnp
from jax import lax
from jax.experimental import pallas as pl
from jax.experimental.pallas import tpu as pltpu
from jax.experimental.pallas import tpu_sc as plsc

_INFO = plsc.get_sparse_core_info()
_NC = _INFO.num_cores
_NS = _INFO.num_subcores
_NW = _NC * _NS

_CHUNK = 1024


def _gather_body(n_per_w, n_chunks, d, x_hbm, w_hbm, out_hbm,
                 idx_v, rows_v, sem):
  wid = lax.axis_index("s") * _NC + lax.axis_index("c")
  w_base = wid * n_per_w

  def step(i, carry):
    base = w_base + i * _CHUNK
    pltpu.sync_copy(x_hbm.at[pl.ds(base, _CHUNK)], idx_v)
    pltpu.async_copy(w_hbm.at[idx_v], rows_v, sem).wait()
    pltpu.sync_copy(rows_v, out_hbm.at[pl.ds(base, _CHUNK)])
    return carry

  lax.fori_loop(0, n_chunks, step, 0)


@functools.partial(jax.jit, static_argnames=("n", "d"))
def _gather(x_flat, w, n, d):
  n_per_w = n // _NW
  n_chunks = n_per_w // _CHUNK
  mesh = plsc.VectorSubcoreMesh(core_axis_name="c", subcore_axis_name="s")
  body = functools.partial(_gather_body, n_per_w, n_chunks, d)
  return pl.kernel(
      body,
      out_type=jax.ShapeDtypeStruct((n, d), jnp.float32),
      mesh=mesh,
      compiler_params=pltpu.CompilerParams(use_tc_tiling_on_sc=False),
      scratch_types=[
          pltpu.VMEM((_CHUNK,), jnp.int32),
          pltpu.VMEM((_CHUNK, d), jnp.float32),
          pltpu.SemaphoreType.DMA,
      ],
  )(x_flat, w)


def kernel(x, W):
  b, h = x.shape
  _, d = W.shape
  n = b * h
  x_flat = x.reshape(n).astype(jnp.int32)
  out = _gather(x_flat, W, n, d)
  return out.reshape(b, h, d)


# speedup vs baseline: 5.0373x; 1.0475x over previous
import functools

import jax
import jax.numpy as jnp
from jax import lax
from jax.experimental import pallas as pl
from jax.experimental.pallas import tpu as pltpu
from jax.experimental.pallas import tpu_sc as plsc

_INFO = plsc.get_sparse_core_info()
_NC = _INFO.num_cores
_NS = _INFO.num_subcores
_NW = _NC * _NS

_CHUNK = 1600


def _gather_body(n_per_w, n_chunks, d, x_hbm, w_hbm, out_hbm,
                 idx0, idx1, rows0, rows1,
                 s_i0, s_i1, s_g0, s_g1, s_s0, s_s1):
  wid = lax.axis_index("s") * _NC + lax.axis_index("c")
  w_base = wid * n_per_w
  idx = (idx0, idx1)
  rows = (rows0, rows1)
  s_i = (s_i0, s_i1)
  s_g = (s_g0, s_g1)
  s_s = (s_s0, s_s1)

  def x_slice(i):
    return x_hbm.at[pl.ds(w_base + i * _CHUNK, _CHUNK)]

  def out_slice(i):
    return out_hbm.at[pl.ds(w_base + i * _CHUNK, _CHUNK)]

  pltpu.async_copy(x_slice(0), idx[0], s_i[0])

  def pair(g, carry):
    for b in range(2):
      i = g * 2 + b
      pltpu.make_async_copy(x_slice(i), idx[b], s_i[b]).wait()

      @pl.when(g > 0)
      def _():
        pltpu.make_async_copy(rows[b], out_slice(i), s_s[b]).wait()

      gather = pltpu.async_copy(w_hbm.at[idx[b]], rows[b], s_g[b])
      nxt = lax.rem(i + 1, n_chunks)
      pltpu.async_copy(x_slice(nxt), idx[1 - b], s_i[1 - b])
      gather.wait()

      pltpu.async_copy(rows[b], out_slice(i), s_s[b])
    return carry

  lax.fori_loop(0, n_chunks // 2, pair, 0)

  pltpu.make_async_copy(rows[0], out_slice(n_chunks - 2), s_s[0]).wait()
  pltpu.make_async_copy(rows[1], out_slice(n_chunks - 1), s_s[1]).wait()
  pltpu.make_async_copy(x_slice(0), idx[0], s_i[0]).wait()


@functools.partial(jax.jit, static_argnames=("n", "d"))
def _gather(x_flat, w, n, d):
  n_per_w = n // _NW
  n_chunks = n_per_w // _CHUNK
  mesh = plsc.VectorSubcoreMesh(core_axis_name="c", subcore_axis_name="s")
  body = functools.partial(_gather_body, n_per_w, n_chunks, d)
  return pl.kernel(
      body,
      out_type=jax.ShapeDtypeStruct((n, d), jnp.float32),
      mesh=mesh,
      compiler_params=pltpu.CompilerParams(use_tc_tiling_on_sc=False),
      scratch_types=[
          pltpu.VMEM((_CHUNK,), jnp.int32),
          pltpu.VMEM((_CHUNK,), jnp.int32),
          pltpu.VMEM((_CHUNK, d), jnp.float32),
          pltpu.VMEM((_CHUNK, d), jnp.float32),
          pltpu.SemaphoreType.DMA,
          pltpu.SemaphoreType.DMA,
          pltpu.SemaphoreType.DMA,
          pltpu.SemaphoreType.DMA,
          pltpu.SemaphoreType.DMA,
          pltpu.SemaphoreType.DMA,
      ],
  )(x_flat, w)


def kernel(x, W):
  b, h = x.shape
  _, d = W.shape
  n = b * h
  x_flat = x.reshape(n).astype(jnp.int32)
  out = _gather(x_flat, W, n, d)
  return out.reshape(b, h, d)
